# baseline (speedup 1.0000x reference)
import functools

import jax
import jax.numpy as jnp
from jax.experimental import pallas as pl
from jax.experimental.pallas import tpu as pltpu

_BF = jnp.bfloat16


def _dot_t(a, b):
    return jax.lax.dot_general(
        a, b, (((1,), (1,)), ((), ())), preferred_element_type=jnp.float32
    )


def _dot(a, b):
    return jax.lax.dot_general(
        a, b, (((1,), (0,)), ((), ())), preferred_element_type=jnp.float32
    )


def _stage_a_kernel(x_ref, gw_ref, rl_ref, xb_ref):
    x = x_ref[...]
    rl_ref[...] = _dot_t(x, gw_ref[...])
    xb_ref[...] = x.astype(_BF)


def _gate_up_kernel(nW, tW, nF, tF, xb_ref, w1_ref, w3_ref, u1_ref, u3_ref,
                    v1_ref, v3_ref, h_ref, m1_s, m3_s):
    i = pl.program_id(0)

    @pl.when(i < nW)
    def _merge():
        m1_s[pl.ds(i * tW, tW), :] = (
            w1_ref[...] + _dot(u1_ref[...].astype(_BF), v1_ref[...])
        ).astype(_BF)
        m3_s[pl.ds(i * tW, tW), :] = (
            w3_ref[...] + _dot(u3_ref[...].astype(_BF), v3_ref[...])
        ).astype(_BF)

    @pl.when(i >= nW)
    def _compute():
        f = (i - nW) % nF
        xb = xb_ref[...]
        gate = _dot_t(xb, m1_s[pl.ds(f * tF, tF), :])
        up = _dot_t(xb, m3_s[pl.ds(f * tF, tF), :])
        h_ref[...] = (jax.nn.silu(gate) * up).astype(_BF)


def _down_kernel(nW, tW, nH, tH, h_ref, w2_ref, u2_ref, v2_ref, o_ref, m2_s):
    i = pl.program_id(0)

    @pl.when(i < nW)
    def _merge():
        m2_s[pl.ds(i * tW, tW), :] = (
            w2_ref[...] + _dot(u2_ref[...].astype(_BF), v2_ref[...])
        ).astype(_BF)

    @pl.when(i >= nW)
    def _compute():
        hh = (i - nW) % nH
        o_ref[...] = _dot_t(h_ref[...], m2_s[pl.ds(hh * tH, tH), :])


def kernel(hidden_states, gate_w, w1, w2, w3, u1, v1, u2, v2, u3, v3):
    b, s, d = hidden_states.shape
    T = b * s
    H = d
    F = w1.shape[0]
    R = u1.shape[1]
    E = gate_w.shape[0]
    x = hidden_states.reshape(T, H)

    v1b, v3b, v2b = v1.astype(_BF), v3.astype(_BF), v2.astype(_BF)

    tMa = min(1024, T)
    nMa = T // tMa
    rl, xb = pl.pallas_call(
        _stage_a_kernel,
        grid=(nMa,),
        in_specs=[
            pl.BlockSpec((tMa, H), lambda m: (m, 0)),
            pl.BlockSpec((E, H), lambda m: (0, 0)),
        ],
        out_specs=[
            pl.BlockSpec((tMa, E), lambda m: (m, 0)),
            pl.BlockSpec((tMa, H), lambda m: (m, 0)),
        ],
        out_shape=[
            jax.ShapeDtypeStruct((T, E), jnp.float32),
            jax.ShapeDtypeStruct((T, H), _BF),
        ],
    )(x, gate_w)

    tM = min(512, T)
    nM = T // tM
    tF = min(1024, F)
    nF = F // tF
    tW = min(256, F)
    nW = F // tW
    nB = nW + nM * nF

    def _b_compute_idx(i):
        j = jnp.maximum(i - nW, 0)
        return j // nF, j % nF

    h = pl.pallas_call(
        functools.partial(_gate_up_kernel, nW, tW, nF, tF),
        grid=(nB,),
        in_specs=[
            pl.BlockSpec((tM, H), lambda i: (_b_compute_idx(i)[0], 0)),
            pl.BlockSpec((tW, H), lambda i: (jnp.minimum(i, nW - 1), 0)),
            pl.BlockSpec((tW, H), lambda i: (jnp.minimum(i, nW - 1), 0)),
            pl.BlockSpec((tW, R), lambda i: (jnp.minimum(i, nW - 1), 0)),
            pl.BlockSpec((tW, R), lambda i: (jnp.minimum(i, nW - 1), 0)),
            pl.BlockSpec((R, H), lambda i: (0, 0)),
            pl.BlockSpec((R, H), lambda i: (0, 0)),
        ],
        out_specs=pl.BlockSpec((tM, tF), lambda i: _b_compute_idx(i)),
        out_shape=jax.ShapeDtypeStruct((T, F), _BF),
        scratch_shapes=[
            pltpu.VMEM((F, H), _BF),
            pltpu.VMEM((F, H), _BF),
        ],
    )(xb, w1, w3, u1, u3, v1b, v3b)

    tH = min(1024, H)
    nH = H // tH
    tW2 = min(512, H)
    nW2 = H // tW2
    nC = nW2 + nM * nH

    def _c_compute_idx(i):
        j = jnp.maximum(i - nW2, 0)
        return j // nH, j % nH

    out = pl.pallas_call(
        functools.partial(_down_kernel, nW2, tW2, nH, tH),
        grid=(nC,),
        in_specs=[
            pl.BlockSpec((tM, F), lambda i: (_c_compute_idx(i)[0], 0)),
            pl.BlockSpec((tW2, F), lambda i: (jnp.minimum(i, nW2 - 1), 0)),
            pl.BlockSpec((tW2, R), lambda i: (jnp.minimum(i, nW2 - 1), 0)),
            pl.BlockSpec((R, F), lambda i: (0, 0)),
        ],
        out_specs=pl.BlockSpec((tM, tH), lambda i: _c_compute_idx(i)),
        out_shape=jax.ShapeDtypeStruct((T, H), jnp.float32),
        scratch_shapes=[
            pltpu.VMEM((H, F), _BF),
        ],
    )(h, w2, u2, v2b)

    return out.reshape(b, s, d), rl


# speedup vs baseline: 1.0004x; 1.0004x over previous
import functools

import jax
import jax.numpy as jnp
from jax.experimental import pallas as pl
from jax.experimental.pallas import tpu as pltpu

_BF = jnp.bfloat16


def _dot_t(a, b):
    return jax.lax.dot_general(
        a, b, (((1,), (1,)), ((), ())), preferred_element_type=jnp.float32
    )


def _dot(a, b):
    return jax.lax.dot_general(
        a, b, (((1,), (0,)), ((), ())), preferred_element_type=jnp.float32
    )


def _merge_kernel(w1_ref, w3_ref, w2_ref, u1_ref, u3_ref, u2_ref,
                  v1_ref, v3_ref, v2_ref, m1_ref, m3_ref, m2_ref):
    m1_ref[...] = (
        w1_ref[...] + _dot(u1_ref[...].astype(_BF), v1_ref[...])
    ).astype(_BF)
    m3_ref[...] = (
        w3_ref[...] + _dot(u3_ref[...].astype(_BF), v3_ref[...])
    ).astype(_BF)
    m2_ref[...] = (
        w2_ref[...] + _dot(u2_ref[...].astype(_BF), v2_ref[...])
    ).astype(_BF)


def _gate_up_kernel(nF, tF, x_ref, gw_ref, m1_ref, m3_ref, rl_ref, h_ref, xb_s):
    f = pl.program_id(1)

    @pl.when(f == 0)
    def _prep():
        x = x_ref[...]
        rl_ref[...] = _dot_t(x, gw_ref[...])
        xb_s[...] = x.astype(_BF)

    xb = xb_s[...]
    gate = _dot_t(xb, m1_ref[pl.ds(f * tF, tF), :])
    up = _dot_t(xb, m3_ref[pl.ds(f * tF, tF), :])
    h_ref[...] = (jax.nn.silu(gate) * up).astype(_BF)


def _down_kernel(nH, tH, h_ref, m2_ref, o_ref):
    hh = pl.program_id(1)
    o_ref[...] = _dot_t(h_ref[...], m2_ref[pl.ds(hh * tH, tH), :])


def kernel(hidden_states, gate_w, w1, w2, w3, u1, v1, u2, v2, u3, v3):
    b, s, d = hidden_states.shape
    T = b * s
    H = d
    F = w1.shape[0]
    R = u1.shape[1]
    E = gate_w.shape[0]
    x = hidden_states.reshape(T, H)

    v1b, v3b, v2b = v1.astype(_BF), v3.astype(_BF), v2.astype(_BF)

    tFm = min(512, F)
    nFm = F // tFm
    tHm = max(H // nFm, 8)
    m1, m3, m2 = pl.pallas_call(
        _merge_kernel,
        grid=(nFm,),
        in_specs=[
            pl.BlockSpec((tFm, H), lambda i: (i, 0)),
            pl.BlockSpec((tFm, H), lambda i: (i, 0)),
            pl.BlockSpec((tHm, F), lambda i: (i, 0)),
            pl.BlockSpec((tFm, R), lambda i: (i, 0)),
            pl.BlockSpec((tFm, R), lambda i: (i, 0)),
            pl.BlockSpec((tHm, R), lambda i: (i, 0)),
            pl.BlockSpec((R, H), lambda i: (0, 0)),
            pl.BlockSpec((R, H), lambda i: (0, 0)),
            pl.BlockSpec((R, F), lambda i: (0, 0)),
        ],
        out_specs=[
            pl.BlockSpec((tFm, H), lambda i: (i, 0)),
            pl.BlockSpec((tFm, H), lambda i: (i, 0)),
            pl.BlockSpec((tHm, F), lambda i: (i, 0)),
        ],
        out_shape=[
            jax.ShapeDtypeStruct((F, H), _BF),
            jax.ShapeDtypeStruct((F, H), _BF),
            jax.ShapeDtypeStruct((H, F), _BF),
        ],
    )(w1, w3, w2, u1, u3, u2, v1b, v3b, v2b)

    tM = min(512, T)
    nM = T // tM
    tF = min(2048, F)
    nF = F // tF
    h = pl.pallas_call(
        functools.partial(_gate_up_kernel, nF, tF),
        grid=(nM, nF),
        in_specs=[
            pl.BlockSpec((tM, H), lambda m, f: (m, 0)),
            pl.BlockSpec((E, H), lambda m, f: (0, 0)),
            pl.BlockSpec((F, H), lambda m, f: (0, 0)),
            pl.BlockSpec((F, H), lambda m, f: (0, 0)),
        ],
        out_specs=[
            pl.BlockSpec((tM, E), lambda m, f: (m, 0)),
            pl.BlockSpec((tM, tF), lambda m, f: (m, f)),
        ],
        out_shape=[
            jax.ShapeDtypeStruct((T, E), jnp.float32),
            jax.ShapeDtypeStruct((T, F), _BF),
        ],
        scratch_shapes=[
            pltpu.VMEM((tM, H), _BF),
        ],
    )(x, gate_w, m1, m3)
    rl, h = h[0], h[1]

    tMc = min(1024, T)
    nMc = T // tMc
    tH = min(1024, H)
    nH = H // tH
    out = pl.pallas_call(
        functools.partial(_down_kernel, nH, tH),
        grid=(nMc, nH),
        in_specs=[
            pl.BlockSpec((tMc, F), lambda m, hh: (m, 0)),
            pl.BlockSpec((H, F), lambda m, hh: (0, 0)),
        ],
        out_specs=pl.BlockSpec((tMc, tH), lambda m, hh: (m, hh)),
        out_shape=jax.ShapeDtypeStruct((T, H), jnp.float32),
    )(h, m2)

    return out.reshape(b, s, d), rl


# speedup vs baseline: 1.0189x; 1.0185x over previous
import jax
import jax.numpy as jnp
from jax.experimental import pallas as pl
from jax.experimental.pallas import tpu as pltpu

_BF = jnp.bfloat16


def _dot_t(a, b):
    return jax.lax.dot_general(
        a, b, (((1,), (1,)), ((), ())), preferred_element_type=jnp.float32
    )


def _dot(a, b):
    return jax.lax.dot_general(
        a, b, (((1,), (0,)), ((), ())), preferred_element_type=jnp.float32
    )


def _stage_a_kernel(x_ref, gw_ref, rl_ref, xb_ref):
    x = x_ref[...]
    rl_ref[...] = _dot_t(x, gw_ref[...])
    xb_ref[...] = x.astype(_BF)


def _gate_up_kernel(xb_ref, w1_ref, w3_ref, u1_ref, u3_ref, v1_ref, v3_ref,
                    h_ref, m1_ref, m3_ref):
    @pl.when(pl.program_id(1) == 0)
    def _merge():
        v1b = v1_ref[...].astype(_BF)
        v3b = v3_ref[...].astype(_BF)
        m1_ref[...] = (
            w1_ref[...] + _dot(u1_ref[...].astype(_BF), v1b)
        ).astype(_BF)
        m3_ref[...] = (
            w3_ref[...] + _dot(u3_ref[...].astype(_BF), v3b)
        ).astype(_BF)

    xb = xb_ref[...]
    gate = _dot_t(xb, m1_ref[...])
    up = _dot_t(xb, m3_ref[...])
    h_ref[...] = (jax.nn.silu(gate) * up).astype(_BF)


def _down_kernel(h_ref, w2_ref, u2_ref, v2_ref, o_ref, m2_ref):
    @pl.when(pl.program_id(1) == 0)
    def _merge():
        m2_ref[...] = (
            w2_ref[...] + _dot(u2_ref[...].astype(_BF), v2_ref[...].astype(_BF))
        ).astype(_BF)

    o_ref[...] = _dot_t(h_ref[...], m2_ref[...])


def kernel(hidden_states, gate_w, w1, w2, w3, u1, v1, u2, v2, u3, v3):
    b, s, d = hidden_states.shape
    T = b * s
    H = d
    F = w1.shape[0]
    R = u1.shape[1]
    E = gate_w.shape[0]
    x = hidden_states.reshape(T, H)

    tMa = min(1024, T)
    nMa = T // tMa

    rl, xb = pl.pallas_call(
        _stage_a_kernel,
        grid=(nMa,),
        in_specs=[
            pl.BlockSpec((tMa, H), lambda m: (m, 0)),
            pl.BlockSpec((E, H), lambda m: (0, 0)),
        ],
        out_specs=[
            pl.BlockSpec((tMa, E), lambda m: (m, 0)),
            pl.BlockSpec((tMa, H), lambda m: (m, 0)),
        ],
        out_shape=[
            jax.ShapeDtypeStruct((T, E), jnp.float32),
            jax.ShapeDtypeStruct((T, H), _BF),
        ],
    )(x, gate_w)

    tM = min(1024, T)
    nM = T // tM
    tF = min(512, F)
    nF = F // tF
    h = pl.pallas_call(
        _gate_up_kernel,
        grid=(nF, nM),
        in_specs=[
            pl.BlockSpec((tM, H), lambda f, m: (m, 0)),
            pl.BlockSpec((tF, H), lambda f, m: (f, 0)),
            pl.BlockSpec((tF, H), lambda f, m: (f, 0)),
            pl.BlockSpec((tF, R), lambda f, m: (f, 0)),
            pl.BlockSpec((tF, R), lambda f, m: (f, 0)),
            pl.BlockSpec((R, H), lambda f, m: (0, 0)),
            pl.BlockSpec((R, H), lambda f, m: (0, 0)),
        ],
        out_specs=pl.BlockSpec((tM, tF), lambda f, m: (m, f)),
        out_shape=jax.ShapeDtypeStruct((T, F), _BF),
        scratch_shapes=[
            pltpu.VMEM((tF, H), _BF),
            pltpu.VMEM((tF, H), _BF),
        ],
    )(xb, w1, w3, u1, u3, v1, v3)

    tH = min(512, H)
    nH = H // tH
    out = pl.pallas_call(
        _down_kernel,
        grid=(nH, nM),
        in_specs=[
            pl.BlockSpec((tM, F), lambda hh, m: (m, 0)),
            pl.BlockSpec((tH, F), lambda hh, m: (hh, 0)),
            pl.BlockSpec((tH, R), lambda hh, m: (hh, 0)),
            pl.BlockSpec((R, F), lambda hh, m: (0, 0)),
        ],
        out_specs=pl.BlockSpec((tM, tH), lambda hh, m: (m, hh)),
        out_shape=jax.ShapeDtypeStruct((T, H), jnp.float32),
        scratch_shapes=[
            pltpu.VMEM((tH, F), _BF),
        ],
    )(h, w2, u2, v2)

    return out.reshape(b, s, d), rl


# speedup vs baseline: 1.0210x; 1.0020x over previous
import jax
import jax.numpy as jnp
from jax.experimental import pallas as pl
from jax.experimental.pallas import tpu as pltpu

_BF = jnp.bfloat16


def _dot_t(a, b):
    return jax.lax.dot_general(
        a, b, (((1,), (1,)), ((), ())), preferred_element_type=jnp.float32
    )


def _dot(a, b):
    return jax.lax.dot_general(
        a, b, (((1,), (0,)), ((), ())), preferred_element_type=jnp.float32
    )


def _stage_a_kernel(x_ref, gw_ref, rl_ref, xb_ref):
    x = x_ref[...]
    rl_ref[...] = _dot_t(x, gw_ref[...])
    xb_ref[...] = x.astype(_BF)


def _gate_up_kernel(xb_ref, w1_ref, w3_ref, u1_ref, u3_ref, v1_ref, v3_ref,
                    h_ref, m1_ref, m3_ref):
    @pl.when(pl.program_id(1) == 0)
    def _merge():
        v1b = v1_ref[...].astype(_BF)
        v3b = v3_ref[...].astype(_BF)
        m1_ref[...] = (
            w1_ref[...] + _dot(u1_ref[...].astype(_BF), v1b)
        ).astype(_BF)
        m3_ref[...] = (
            w3_ref[...] + _dot(u3_ref[...].astype(_BF), v3b)
        ).astype(_BF)

    xb = xb_ref[...]
    gate = _dot_t(xb, m1_ref[...])
    up = _dot_t(xb, m3_ref[...])
    h_ref[...] = (jax.nn.silu(gate) * up).astype(_BF)


def _down_kernel(h_ref, w2_ref, u2_ref, v2_ref, o_ref, m2_ref):
    @pl.when(pl.program_id(1) == 0)
    def _merge():
        m2_ref[...] = (
            w2_ref[...] + _dot(u2_ref[...].astype(_BF), v2_ref[...].astype(_BF))
        ).astype(_BF)

    o_ref[...] = _dot_t(h_ref[...], m2_ref[...])


def kernel(hidden_states, gate_w, w1, w2, w3, u1, v1, u2, v2, u3, v3):
    b, s, d = hidden_states.shape
    T = b * s
    H = d
    F = w1.shape[0]
    R = u1.shape[1]
    E = gate_w.shape[0]
    x = hidden_states.reshape(T, H)

    tMa = min(1024, T)
    nMa = T // tMa

    rl, xb = pl.pallas_call(
        _stage_a_kernel,
        grid=(nMa,),
        in_specs=[
            pl.BlockSpec((tMa, H), lambda m: (m, 0)),
            pl.BlockSpec((E, H), lambda m: (0, 0)),
        ],
        out_specs=[
            pl.BlockSpec((tMa, E), lambda m: (m, 0)),
            pl.BlockSpec((tMa, H), lambda m: (m, 0)),
        ],
        out_shape=[
            jax.ShapeDtypeStruct((T, E), jnp.float32),
            jax.ShapeDtypeStruct((T, H), _BF),
        ],
    )(x, gate_w)

    tMb = min(1024, T)
    nMb = T // tMb
    tM = min(1024, T)
    nM = T // tM
    tF = min(512, F)
    nF = F // tF
    h = pl.pallas_call(
        _gate_up_kernel,
        grid=(nF, nMb),
        in_specs=[
            pl.BlockSpec((tMb, H), lambda f, m: (m, 0)),
            pl.BlockSpec((tF, H), lambda f, m: (f, 0)),
            pl.BlockSpec((tF, H), lambda f, m: (f, 0)),
            pl.BlockSpec((tF, R), lambda f, m: (f, 0)),
            pl.BlockSpec((tF, R), lambda f, m: (f, 0)),
            pl.BlockSpec((R, H), lambda f, m: (0, 0)),
            pl.BlockSpec((R, H), lambda f, m: (0, 0)),
        ],
        out_specs=pl.BlockSpec((tMb, tF), lambda f, m: (m, f)),
        out_shape=jax.ShapeDtypeStruct((T, F), _BF),
        scratch_shapes=[
            pltpu.VMEM((tF, H), _BF),
            pltpu.VMEM((tF, H), _BF),
        ],
    )(xb, w1, w3, u1, u3, v1, v3)

    tH = min(512, H)
    nH = H // tH
    out = pl.pallas_call(
        _down_kernel,
        grid=(nH, nM),
        in_specs=[
            pl.BlockSpec((tM, F), lambda hh, m: (m, 0)),
            pl.BlockSpec((tH, F), lambda hh, m: (hh, 0)),
            pl.BlockSpec((tH, R), lambda hh, m: (hh, 0)),
            pl.BlockSpec((R, F), lambda hh, m: (0, 0)),
        ],
        out_specs=pl.BlockSpec((tM, tH), lambda hh, m: (m, hh)),
        out_shape=jax.ShapeDtypeStruct((T, H), jnp.float32),
        scratch_shapes=[
            pltpu.VMEM((tH, F), _BF),
        ],
    )(h, w2, u2, v2)

    return out.reshape(b, s, d), rl
